# baseline (speedup 1.0000x reference)
import functools

import jax
import jax.numpy as jnp
import numpy as np
from jax import lax
from jax.experimental import pallas as pl
from jax.experimental.pallas import tpu as pltpu
from jax.experimental.pallas import tpu_sc as plsc

B = 4096
L = 200
LP = 208
D = 64
NC = 2
NS = 16
NW = NC * NS
ROWS = B // NW
CHUNK0 = 128
CHUNK1 = L - CHUNK0
NBLK = LP // 16

_BCAST_DNUMS = lax.GatherDimensionNumbers(
    offset_dims=(), collapsed_slice_dims=(0,), start_index_map=(0,))


def _lane_gather(vec, idx):
  return lax.gather(vec, idx.reshape(16, 1), _BCAST_DNUMS, (1,),
                    mode=lax.GatherScatterMode.PROMISE_IN_BOUNDS)


def _allreduce_sum(vec, iota_vec):
  for k in (8, 4, 2, 1):
    vec = vec + _lane_gather(vec, iota_vec ^ k)
  return vec


def _sc_body(uidx_hbm, hidx_hbm, wm_hbm, iota_hbm, utab_flat, itab_hbm,
             uout_flat, pout_hbm,
             uidx_v, uginx_v, ucols_v, idxh_v, wm_v, iota_v, rows_v, out_v,
             sem_u, sem0, sem1):
  wid = lax.axis_index("s") * NC + lax.axis_index("c")
  base = wid * ROWS

  pltpu.sync_copy(uidx_hbm.at[pl.ds(base, ROWS)], uidx_v)
  pltpu.sync_copy(iota_hbm, iota_v)
  iv0 = iota_v[...]

  def build_uginx(j, _):
    vec = uidx_v[pl.ds(j * 16, 16)]
    zv = iv0 * 0
    for t in range(16):
      i = j * 16 + t
      idxb = _lane_gather(vec, zv + t)
      for k in range(4):
        doff = iv0 * 1000000 + (k * 16 * 1000000)
        uginx_v[pl.ds(i * D + k * 16, 16)] = idxb + doff
    return 0

  lax.fori_loop(0, ROWS // 16, build_uginx, 0)

  def issue_user(c, _):
    pltpu.async_copy(utab_flat.at[uginx_v.at[pl.ds(c * 128, 128)]],
                     ucols_v.at[pl.ds(c * 128, 128)], sem_u)
    return 0

  lax.fori_loop(0, ROWS * D // 128, issue_user, 0)

  pltpu.sync_copy(hidx_hbm.at[pl.ds(base, ROWS)], idxh_v)
  pltpu.sync_copy(wm_hbm.at[pl.ds(base, ROWS)], wm_v)

  pltpu.make_async_copy(utab_flat.at[pl.ds(0, ROWS * D)], ucols_v,
                        sem_u).wait()
  pltpu.sync_copy(ucols_v, uout_flat.at[pl.ds(base * D, ROWS * D)])

  sems = (sem0, sem1)

  def issue(r, buf, sem):
    pltpu.async_copy(itab_hbm.at[idxh_v.at[r, pl.ds(0, CHUNK0)]],
                     rows_v.at[buf, pl.ds(0, CHUNK0)], sem)
    pltpu.async_copy(itab_hbm.at[idxh_v.at[r, pl.ds(CHUNK0, CHUNK1)]],
                     rows_v.at[buf, pl.ds(CHUNK0, CHUNK1)], sem)

  def wait(r, buf, sem):
    pltpu.make_async_copy(itab_hbm.at[idxh_v.at[r, pl.ds(0, CHUNK0)]],
                          rows_v.at[buf, pl.ds(0, CHUNK0)], sem).wait()
    pltpu.make_async_copy(itab_hbm.at[idxh_v.at[r, pl.ds(CHUNK0, CHUNK1)]],
                          rows_v.at[buf, pl.ds(CHUNK0, CHUNK1)], sem).wait()

  def accum(r, buf):
    def block(j, carry):
      a0, a1, a2, a3, swv = carry
      zv = iota_v[...] * 0
      cw = wm_v[r, pl.ds(j * 16, 16)]
      swv = swv + cw
      for t in range(16):
        c = _lane_gather(cw, zv + t)
        l = j * 16 + t
        a0 = a0 + rows_v[buf, l, pl.ds(0, 16)] * c
        a1 = a1 + rows_v[buf, l, pl.ds(16, 16)] * c
        a2 = a2 + rows_v[buf, l, pl.ds(32, 16)] * c
        a3 = a3 + rows_v[buf, l, pl.ds(48, 16)] * c
      return a0, a1, a2, a3, swv

    z = jnp.zeros((16,), jnp.float32)
    a0, a1, a2, a3, swv = lax.fori_loop(0, NBLK - 1, block, (z, z, z, z, z))
    iv = iota_v[...]
    zv = iv * 0
    cw = wm_v[r, pl.ds((NBLK - 1) * 16, 16)]
    swv = swv + cw
    for t in range(8):
      c = _lane_gather(cw, zv + t)
      l = (NBLK - 1) * 16 + t
      a0 = a0 + rows_v[buf, l, pl.ds(0, 16)] * c
      a1 = a1 + rows_v[buf, l, pl.ds(16, 16)] * c
      a2 = a2 + rows_v[buf, l, pl.ds(32, 16)] * c
      a3 = a3 + rows_v[buf, l, pl.ds(48, 16)] * c

    inv = 1.0 / (_allreduce_sum(swv, iv) + 1e-8)
    out_v[r, pl.ds(0, 16)] = a0 * inv
    out_v[r, pl.ds(16, 16)] = a1 * inv
    out_v[r, pl.ds(32, 16)] = a2 * inv
    out_v[r, pl.ds(48, 16)] = a3 * inv

  issue(0, 0, sem0)
  issue(1, 1, sem1)

  def outer(i, _):
    for b in range(2):
      r = 2 * i + b
      wait(r, b, sems[b])
      accum(r, b)

      @pl.when(r + 2 < ROWS)
      def _():
        issue(r + 2, b, sems[b])
    return 0

  lax.fori_loop(0, ROWS // 2, outer, 0)
  pltpu.sync_copy(out_v, pout_hbm.at[pl.ds(base, ROWS)])


def _sc_pool(user_indices, hist_indices, wm_padded, lane_iota,
             user_table_flat, item_table):
  mesh = plsc.VectorSubcoreMesh(core_axis_name="c", subcore_axis_name="s")
  f = pl.kernel(
      _sc_body,
      out_type=(jax.ShapeDtypeStruct((B * D,), jnp.float32),
                jax.ShapeDtypeStruct((B, D), jnp.float32)),
      mesh=mesh,
      scratch_types=[
          pltpu.VMEM((ROWS,), jnp.int32),
          pltpu.VMEM((ROWS * D,), jnp.int32),
          pltpu.VMEM((ROWS * D,), jnp.float32),
          pltpu.VMEM((ROWS, L), jnp.int32),
          pltpu.VMEM((ROWS, LP), jnp.float32),
          pltpu.VMEM((16,), jnp.int32),
          pltpu.VMEM((2, L, D), jnp.float32),
          pltpu.VMEM((ROWS, D), jnp.float32),
          pltpu.SemaphoreType.DMA,
          pltpu.SemaphoreType.DMA,
          pltpu.SemaphoreType.DMA,
      ],
      compiler_params=pltpu.CompilerParams(use_tc_tiling_on_sc=False),
  )
  return f(user_indices, hist_indices, wm_padded, lane_iota, user_table_flat,
           item_table)


def _mlp_body(u_ref, p_ref, w1_ref, b1_ref, g_ref, be_ref, w2_ref, b2_ref,
              o_ref):
  xt = lax.dot_general(w1_ref[:, :D], u_ref[...], (((1,), (1,)), ((), ())),
                       preferred_element_type=jnp.float32,
                       precision=lax.Precision.HIGHEST)
  xt = xt + lax.dot_general(w1_ref[:, D:], p_ref[...],
                            (((1,), (1,)), ((), ())),
                            preferred_element_type=jnp.float32,
                            precision=lax.Precision.HIGHEST)
  xt = xt + b1_ref[...]
  mean = jnp.mean(xt, axis=1, keepdims=True)
  xc = xt - mean
  var = jnp.mean(xc * xc, axis=1, keepdims=True)
  xt = xc * lax.rsqrt(var + 1e-5)
  xt = xt * g_ref[...] + be_ref[...]
  xt = jnp.maximum(xt, 0.0)
  yt = lax.dot_general(w2_ref[...], xt, (((1,), (0,)), ((), ())),
                       preferred_element_type=jnp.float32,
                       precision=lax.Precision.HIGHEST)
  yt = yt + b2_ref[...]
  nrm = jnp.sqrt(jnp.sum(yt * yt, axis=0, keepdims=True))
  o_ref[...] = yt / jnp.maximum(nrm, 1e-12)


def _mlp(u_emb, pooled, W1, b1, gamma, beta, W2, b2):
  return pl.pallas_call(
      _mlp_body,
      out_shape=jax.ShapeDtypeStruct((D, B), jnp.float32),
  )(u_emb, pooled, W1, b1.reshape(-1, 1), gamma.reshape(-1, 1),
    beta.reshape(-1, 1), W2, b2.reshape(-1, 1))


def kernel(user_indices, hist_indices, hist_weights, hist_mask,
           user_table, item_table, W1, b1, gamma, beta, W2, b2):
  wm = hist_weights * hist_mask
  wm_padded = jnp.pad(wm, ((0, 0), (0, LP - L)))
  lane_iota = jnp.arange(16, dtype=jnp.int32)
  u_flat, pooled = _sc_pool(user_indices, hist_indices, wm_padded, lane_iota,
                            user_table.T.reshape(-1), item_table)
  u_emb = u_flat.reshape(B, D)
  out_t = _mlp(u_emb, pooled, W1, b1, gamma, beta, W2, b2)
  return out_t.T


# speedup vs baseline: 4.5655x; 4.5655x over previous
import functools

import jax
import jax.numpy as jnp
import numpy as np
from jax import lax
from jax.experimental import pallas as pl
from jax.experimental.pallas import tpu as pltpu
from jax.experimental.pallas import tpu_sc as plsc

B = 4096
L = 200
LP = 208
D = 64
NC = 2
NS = 16
NW = NC * NS
ROWS = B // NW
CHUNK0 = 128
CHUNK1 = L - CHUNK0
NBLK = LP // 16

_BCAST_DNUMS = lax.GatherDimensionNumbers(
    offset_dims=(), collapsed_slice_dims=(0,), start_index_map=(0,))


def _lane_gather(vec, idx):
  return lax.gather(vec, idx.reshape(16, 1), _BCAST_DNUMS, (1,),
                    mode=lax.GatherScatterMode.PROMISE_IN_BOUNDS)


def _allreduce_sum(vec, iota_vec):
  for k in (8, 4, 2, 1):
    vec = vec + _lane_gather(vec, iota_vec ^ k)
  return vec


def _sc_body(uidx_hbm, hidx_hbm, wm_hbm, iota_hbm, utab_hbm, itab_hbm,
             uout_hbm, pout_hbm,
             uidx_v, urows_v, idxh_v, wm_v, iota_v,
             rows_v, out_v, sem_u, sem0, sem1):
  wid = lax.axis_index("s") * NC + lax.axis_index("c")
  base = wid * ROWS

  pltpu.sync_copy(uidx_hbm.at[pl.ds(base, ROWS)], uidx_v)
  pltpu.async_copy(utab_hbm.at[uidx_v], urows_v, sem_u)

  pltpu.sync_copy(hidx_hbm.at[pl.ds(base, ROWS)], idxh_v)
  pltpu.sync_copy(wm_hbm.at[pl.ds(base, ROWS)], wm_v)
  pltpu.sync_copy(iota_hbm, iota_v)

  pltpu.make_async_copy(utab_hbm.at[uidx_v], urows_v, sem_u).wait()
  pltpu.sync_copy(urows_v, uout_hbm.at[pl.ds(base, ROWS)])

  sems = (sem0, sem1)

  def issue(r, buf, sem):
    pltpu.async_copy(itab_hbm.at[idxh_v.at[r, pl.ds(0, CHUNK0)]],
                     rows_v.at[buf, pl.ds(0, CHUNK0)], sem)
    pltpu.async_copy(itab_hbm.at[idxh_v.at[r, pl.ds(CHUNK0, CHUNK1)]],
                     rows_v.at[buf, pl.ds(CHUNK0, CHUNK1)], sem)

  def wait(r, buf, sem):
    pltpu.make_async_copy(itab_hbm.at[idxh_v.at[r, pl.ds(0, CHUNK0)]],
                          rows_v.at[buf, pl.ds(0, CHUNK0)], sem).wait()
    pltpu.make_async_copy(itab_hbm.at[idxh_v.at[r, pl.ds(CHUNK0, CHUNK1)]],
                          rows_v.at[buf, pl.ds(CHUNK0, CHUNK1)], sem).wait()

  def accum(r, buf):
    def block(j, carry):
      a0, a1, a2, a3, swv = carry
      zv = iota_v[...] * 0
      cw = wm_v[r, pl.ds(j * 16, 16)]
      swv = swv + cw
      for t in range(16):
        c = _lane_gather(cw, zv + t)
        l = j * 16 + t
        a0 = a0 + rows_v[buf, l, pl.ds(0, 16)] * c
        a1 = a1 + rows_v[buf, l, pl.ds(16, 16)] * c
        a2 = a2 + rows_v[buf, l, pl.ds(32, 16)] * c
        a3 = a3 + rows_v[buf, l, pl.ds(48, 16)] * c
      return a0, a1, a2, a3, swv

    z = jnp.zeros((16,), jnp.float32)
    a0, a1, a2, a3, swv = lax.fori_loop(0, NBLK - 1, block, (z, z, z, z, z))
    iv = iota_v[...]
    zv = iv * 0
    cw = wm_v[r, pl.ds((NBLK - 1) * 16, 16)]
    swv = swv + cw
    for t in range(8):
      c = _lane_gather(cw, zv + t)
      l = (NBLK - 1) * 16 + t
      a0 = a0 + rows_v[buf, l, pl.ds(0, 16)] * c
      a1 = a1 + rows_v[buf, l, pl.ds(16, 16)] * c
      a2 = a2 + rows_v[buf, l, pl.ds(32, 16)] * c
      a3 = a3 + rows_v[buf, l, pl.ds(48, 16)] * c

    inv = 1.0 / (_allreduce_sum(swv, iv) + 1e-8)
    out_v[r, pl.ds(0, 16)] = a0 * inv
    out_v[r, pl.ds(16, 16)] = a1 * inv
    out_v[r, pl.ds(32, 16)] = a2 * inv
    out_v[r, pl.ds(48, 16)] = a3 * inv

  issue(0, 0, sem0)
  issue(1, 1, sem1)

  def outer(i, _):
    for b in range(2):
      r = 2 * i + b
      wait(r, b, sems[b])
      accum(r, b)

      @pl.when(r + 2 < ROWS)
      def _():
        issue(r + 2, b, sems[b])
    return 0

  lax.fori_loop(0, ROWS // 2, outer, 0)
  pltpu.sync_copy(out_v, pout_hbm.at[pl.ds(base, ROWS)])


def _sc_pool(user_indices, hist_indices, wm_padded, lane_iota,
             user_table_t, item_table):
  mesh = plsc.VectorSubcoreMesh(core_axis_name="c", subcore_axis_name="s")
  f = pl.kernel(
      _sc_body,
      out_type=(jax.ShapeDtypeStruct((B, D), jnp.float32),
                jax.ShapeDtypeStruct((B, D), jnp.float32)),
      mesh=mesh,
      scratch_types=[
          pltpu.VMEM((ROWS,), jnp.int32),
          pltpu.VMEM((ROWS, D), jnp.float32),
          pltpu.VMEM((ROWS, L), jnp.int32),
          pltpu.VMEM((ROWS, LP), jnp.float32),
          pltpu.VMEM((16,), jnp.int32),
          pltpu.VMEM((2, L, D), jnp.float32),
          pltpu.VMEM((ROWS, D), jnp.float32),
          pltpu.SemaphoreType.DMA,
          pltpu.SemaphoreType.DMA,
          pltpu.SemaphoreType.DMA,
      ],
      compiler_params=pltpu.CompilerParams(use_tc_tiling_on_sc=False),
  )
  return f(user_indices, hist_indices, wm_padded, lane_iota, user_table_t,
           item_table)


def _mlp_body(u_ref, p_ref, w1_ref, b1_ref, g_ref, be_ref, w2_ref, b2_ref,
              o_ref):
  xt = lax.dot_general(w1_ref[:, :D], u_ref[...], (((1,), (1,)), ((), ())),
                       preferred_element_type=jnp.float32,
                       precision=lax.Precision.HIGHEST)
  xt = xt + lax.dot_general(w1_ref[:, D:], p_ref[...],
                            (((1,), (1,)), ((), ())),
                            preferred_element_type=jnp.float32,
                            precision=lax.Precision.HIGHEST)
  xt = xt + b1_ref[...]
  mean = jnp.mean(xt, axis=1, keepdims=True)
  xc = xt - mean
  var = jnp.mean(xc * xc, axis=1, keepdims=True)
  xt = xc * lax.rsqrt(var + 1e-5)
  xt = xt * g_ref[...] + be_ref[...]
  xt = jnp.maximum(xt, 0.0)
  yt = lax.dot_general(w2_ref[...], xt, (((1,), (0,)), ((), ())),
                       preferred_element_type=jnp.float32,
                       precision=lax.Precision.HIGHEST)
  yt = yt + b2_ref[...]
  nrm = jnp.sqrt(jnp.sum(yt * yt, axis=0, keepdims=True))
  o_ref[...] = yt / jnp.maximum(nrm, 1e-12)


def _mlp(u_emb, pooled, W1, b1, gamma, beta, W2, b2):
  return pl.pallas_call(
      _mlp_body,
      out_shape=jax.ShapeDtypeStruct((D, B), jnp.float32),
  )(u_emb, pooled, W1, b1.reshape(-1, 1), gamma.reshape(-1, 1),
    beta.reshape(-1, 1), W2, b2.reshape(-1, 1))


def kernel(user_indices, hist_indices, hist_weights, hist_mask,
           user_table, item_table, W1, b1, gamma, beta, W2, b2):
  wm = hist_weights * hist_mask
  wm_padded = jnp.pad(wm, ((0, 0), (0, LP - L)))
  lane_iota = jnp.arange(16, dtype=jnp.int32)
  u_emb, pooled = _sc_pool(user_indices, hist_indices, wm_padded, lane_iota,
                           user_table, item_table)
  out_t = _mlp(u_emb, pooled, W1, b1, gamma, beta, W2, b2)
  return out_t.T
